# baseline (speedup 1.0000x reference)
import functools

import jax
import jax.numpy as jnp
from jax import lax
from jax.experimental import pallas as pl
from jax.experimental.pallas import tpu as pltpu
from jax.experimental.pallas import tpu_sc as plsc

_B, _T, _D, _H = 64, 8, 4096, 4096
_EPS = 1e-5
_KC = 512
_C = _H // _KC

_NNZ = 167772
_NT = 16
_CHUNK_ROWS = 82
_CHUNK = _CHUNK_ROWS * 128
_NNZ_PAD = _NT * _CHUNK
_SLAB_ROWS = 256
_SLAB_ELEMS = _SLAB_ROWS * _H
_SLABS_PER_SC = _H // (2 * _SLAB_ROWS)
_STRIPE = _SLAB_ELEMS // _NT


def _pad_coo(idx, val):
    pad = _NNZ_PAD - _NNZ
    maj = jnp.concatenate([idx[1], jnp.zeros((pad,), jnp.int32)])
    mnr = jnp.concatenate([idx[0], jnp.zeros((pad,), jnp.int32)])
    v = jnp.concatenate([val, jnp.zeros((pad,), jnp.float32)])
    return maj, mnr, v


def _densify_body(maj_ih, mnr_ih, val_ih, maj_hh, mnr_hh, val_hh, zeros_hbm,
                  wt_ih, wt_hh, pre_v, mnr_v, val_v, off_b, v_b, slab):
    core = lax.axis_index("c")
    sid = lax.axis_index("s")
    core_base = core * (_SLABS_PER_SC * _SLAB_ELEMS)

    for (maj_h, mnr_h, val_h, wt_h) in (
        (maj_ih, mnr_ih, val_ih, wt_ih),
        (maj_hh, mnr_hh, val_hh, wt_hh),
    ):
        pltpu.sync_copy(maj_h.at[pl.ds(sid * _CHUNK, _CHUNK)], pre_v)
        pltpu.sync_copy(mnr_h.at[pl.ds(sid * _CHUNK, _CHUNK)], mnr_v)
        pltpu.sync_copy(val_h.at[pl.ds(sid * _CHUNK, _CHUNK)], val_v)

        def _pre(i, _):
            maj16 = pre_v[pl.ds(i * 16, 16)]
            mnr16 = mnr_v[pl.ds(i * 16, 16)]
            pre_v[pl.ds(i * 16, 16)] = maj16 * _H + mnr16 - core_base
            return 0

        lax.fori_loop(0, _CHUNK // 16, _pre, 0)

        for s in range(_SLABS_PER_SC):
            pltpu.sync_copy(zeros_hbm.at[pl.ds(sid * _STRIPE, _STRIPE)],
                            slab.at[pl.ds(sid * _STRIPE, _STRIPE)])
            plsc.subcore_barrier()

            def _mk(j, _):
                for k in range(8):
                    o = j * 128 + k * 16
                    rel = pre_v[pl.ds(o, 16)] - s * _SLAB_ELEMS
                    inb = plsc.bitcast(rel, jnp.uint32) < _SLAB_ELEMS
                    off_b[pl.ds(o, 16)] = rel & (_SLAB_ELEMS - 1)
                    v_b[pl.ds(o, 16)] = jnp.where(
                        inb, val_v[pl.ds(o, 16)], 0.0)
                return 0

            lax.fori_loop(0, _CHUNK_ROWS, _mk, 0)
            pltpu.sync_copy(v_b, slab.at[off_b], add=True)
            plsc.subcore_barrier()
            base = (core * _SLABS_PER_SC + s) * _SLAB_ELEMS + sid * _STRIPE
            pltpu.sync_copy(slab.at[pl.ds(sid * _STRIPE, _STRIPE)],
                            wt_h.at[pl.ds(base, _STRIPE)])


@jax.jit
def _densify(ih_indices, ih_values, hh_indices, hh_values):
    maj_i, mnr_i, v_i = _pad_coo(ih_indices, ih_values)
    maj_h, mnr_h, v_h = _pad_coo(hh_indices, hh_values)
    zeros = jnp.zeros((_SLAB_ELEMS,), jnp.float32)

    fn = pl.kernel(
        _densify_body,
        out_type=[
            jax.ShapeDtypeStruct((_D * _H,), jnp.float32),
            jax.ShapeDtypeStruct((_H * _H,), jnp.float32),
        ],
        mesh=plsc.VectorSubcoreMesh(
            core_axis_name="c", subcore_axis_name="s"),
        scratch_types=[
            pltpu.VMEM((_CHUNK,), jnp.int32),
            pltpu.VMEM((_CHUNK,), jnp.int32),
            pltpu.VMEM((_CHUNK,), jnp.float32),
            pltpu.VMEM((_CHUNK,), jnp.int32),
            pltpu.VMEM((_CHUNK,), jnp.float32),
            pltpu.VMEM_SHARED((_SLAB_ELEMS,), jnp.float32),
        ],
    )
    wt_ih, wt_hh = fn(maj_i, mnr_i, v_i, maj_h, mnr_h, v_h, zeros)
    return wt_ih.reshape(_D, _H), wt_hh.reshape(_H, _H)


def _ih_matmul_body(x_ref, w_ref, b_ref, out_ref):
    out_ref[...] = (
        jnp.dot(x_ref[...], w_ref[...],
                preferred_element_type=jnp.float32,
                precision=jax.lax.Precision.HIGHEST)
        + b_ref[...]
    )


def _recurrence_body(ih_ref, w_ref, g_ref, bt_ref, out_ref, h_scr, acc):
    t = pl.program_id(0)
    c = pl.program_id(1)

    @pl.when(c == 0)
    def _init():
        acc[...] = ih_ref[0]

    @pl.when(t > 0)
    def _mm():
        acc[...] += jnp.dot(
            h_scr[:, pl.ds(c * _KC, _KC)], w_ref[...],
            preferred_element_type=jnp.float32,
            precision=jax.lax.Precision.HIGHEST)

    @pl.when(c == _C - 1)
    def _ln():
        p = acc[...]
        mu = jnp.mean(p, axis=1, keepdims=True)
        var = jnp.mean((p - mu) * (p - mu), axis=1, keepdims=True)
        hn = jnp.tanh((p - mu) * jax.lax.rsqrt(var + _EPS) * g_ref[...]
                      + bt_ref[...])
        h_scr[...] = hn
        out_ref[0] = hn


def _dense_recurrence(xs, w_ihT, w_hhT, bias, ln_gamma, ln_beta):
    ih_all = pl.pallas_call(
        _ih_matmul_body,
        grid=(_H // 512,),
        in_specs=[
            pl.BlockSpec((_T * _B, _D), lambda j: (0, 0)),
            pl.BlockSpec((_D, 512), lambda j: (0, j)),
            pl.BlockSpec((1, 512), lambda j: (0, j)),
        ],
        out_specs=pl.BlockSpec((_T * _B, 512), lambda j: (0, j)),
        out_shape=jax.ShapeDtypeStruct((_T * _B, _H), jnp.float32),
    )(xs, w_ihT, bias.reshape(1, _H))

    out = pl.pallas_call(
        _recurrence_body,
        grid=(_T, _C),
        in_specs=[
            pl.BlockSpec((1, _B, _H), lambda t, c: (t, 0, 0)),
            pl.BlockSpec((_KC, _H),
                         lambda t, c: (jnp.where(t == 0, 0, c), 0)),
            pl.BlockSpec((1, _H), lambda t, c: (0, 0)),
            pl.BlockSpec((1, _H), lambda t, c: (0, 0)),
        ],
        out_specs=pl.BlockSpec((1, _B, _H), lambda t, c: (t, 0, 0)),
        out_shape=jax.ShapeDtypeStruct((_T, _B, _H), jnp.float32),
        scratch_shapes=[
            pltpu.VMEM((_B, _H), jnp.float32),
            pltpu.VMEM((_B, _H), jnp.float32),
        ],
    )(ih_all.reshape(_T, _B, _H), w_hhT,
      ln_gamma.reshape(1, _H), ln_beta.reshape(1, _H))
    return out.transpose(1, 0, 2)


def kernel(x, ih_indices, ih_values, hh_indices, hh_values,
           bias_ih, bias_hh, ln_gamma, ln_beta):
    w_ihT, w_hhT = _densify(ih_indices, ih_values, hh_indices, hh_values)
    xs = x.transpose(1, 0, 2).reshape(_T * _B, _D)
    bias = bias_ih + bias_hh
    return _dense_recurrence(xs, w_ihT, w_hhT, bias, ln_gamma, ln_beta)


# speedup vs baseline: 1.4420x; 1.4420x over previous
import functools

import jax
import jax.numpy as jnp
from jax import lax
from jax.experimental import pallas as pl
from jax.experimental.pallas import tpu as pltpu
from jax.experimental.pallas import tpu_sc as plsc

_B, _T, _D, _H = 64, 8, 4096, 4096
_EPS = 1e-5

_NNZ = 167772
_NT = 16
_CHUNK_ROWS = 82
_CHUNK = _CHUNK_ROWS * 128
_NNZ_PAD = _NT * _CHUNK
_SLAB_ROWS = 256
_SLAB_ELEMS = _SLAB_ROWS * _H
_SLABS_PER_SC = _H // (2 * _SLAB_ROWS)
_STRIPE = _SLAB_ELEMS // _NT


def _pad_coo(maj, mnr, val):
    pad = _NNZ_PAD - _NNZ
    maj = jnp.concatenate([maj, jnp.zeros((pad,), jnp.int32)])
    mnr = jnp.concatenate([mnr, jnp.zeros((pad,), jnp.int32)])
    v = jnp.concatenate([val, jnp.zeros((pad,), jnp.float32)])
    return maj, mnr, v


def _densify_body(maj_ih, mnr_ih, val_ih, maj_hh, mnr_hh, val_hh, zeros_hbm,
                  wt_ih, w_hh, pre_v, mnr_v, val_v, off_b, v_b, slab):
    core = lax.axis_index("c")
    sid = lax.axis_index("s")
    core_base = core * (_SLABS_PER_SC * _SLAB_ELEMS)

    for (maj_h, mnr_h, val_h, w_out) in (
        (maj_ih, mnr_ih, val_ih, wt_ih),
        (maj_hh, mnr_hh, val_hh, w_hh),
    ):
        pltpu.sync_copy(maj_h.at[pl.ds(sid * _CHUNK, _CHUNK)], pre_v)
        pltpu.sync_copy(mnr_h.at[pl.ds(sid * _CHUNK, _CHUNK)], mnr_v)
        pltpu.sync_copy(val_h.at[pl.ds(sid * _CHUNK, _CHUNK)], val_v)

        def _pre(i, _):
            maj16 = pre_v[pl.ds(i * 16, 16)]
            mnr16 = mnr_v[pl.ds(i * 16, 16)]
            pre_v[pl.ds(i * 16, 16)] = maj16 * _H + mnr16 - core_base
            return 0

        lax.fori_loop(0, _CHUNK // 16, _pre, 0)

        for s in range(_SLABS_PER_SC):
            pltpu.sync_copy(zeros_hbm.at[pl.ds(sid * _STRIPE, _STRIPE)],
                            slab.at[pl.ds(sid * _STRIPE, _STRIPE)])
            plsc.subcore_barrier()

            def _mk(j, _):
                for k in range(8):
                    o = j * 128 + k * 16
                    rel = pre_v[pl.ds(o, 16)] - s * _SLAB_ELEMS
                    inb = plsc.bitcast(rel, jnp.uint32) < _SLAB_ELEMS
                    off_b[pl.ds(o, 16)] = rel & (_SLAB_ELEMS - 1)
                    v_b[pl.ds(o, 16)] = jnp.where(
                        inb, val_v[pl.ds(o, 16)], 0.0)
                return 0

            lax.fori_loop(0, _CHUNK_ROWS, _mk, 0)
            pltpu.sync_copy(v_b, slab.at[off_b], add=True)
            plsc.subcore_barrier()
            base = (core * _SLABS_PER_SC + s) * _SLAB_ELEMS + sid * _STRIPE
            pltpu.sync_copy(slab.at[pl.ds(sid * _STRIPE, _STRIPE)],
                            w_out.at[pl.ds(base, _STRIPE)])


@jax.jit
def _densify(ih_indices, ih_values, hh_indices, hh_values):
    maj_i, mnr_i, v_i = _pad_coo(ih_indices[1], ih_indices[0], ih_values)
    maj_h, mnr_h, v_h = _pad_coo(hh_indices[0], hh_indices[1], hh_values)
    zeros = jnp.zeros((_SLAB_ELEMS,), jnp.float32)

    fn = pl.kernel(
        _densify_body,
        out_type=[
            jax.ShapeDtypeStruct((_D * _H,), jnp.float32),
            jax.ShapeDtypeStruct((_H * _H,), jnp.float32),
        ],
        mesh=plsc.VectorSubcoreMesh(
            core_axis_name="c", subcore_axis_name="s"),
        scratch_types=[
            pltpu.VMEM((_CHUNK,), jnp.int32),
            pltpu.VMEM((_CHUNK,), jnp.int32),
            pltpu.VMEM((_CHUNK,), jnp.float32),
            pltpu.VMEM((_CHUNK,), jnp.int32),
            pltpu.VMEM((_CHUNK,), jnp.float32),
            pltpu.VMEM_SHARED((_SLAB_ELEMS,), jnp.float32),
        ],
    )
    wt_ih, w_hh = fn(maj_i, mnr_i, v_i, maj_h, mnr_h, v_h, zeros)
    return wt_ih.reshape(_D, _H), w_hh.reshape(_H, _H)


def _ih_matmul_body(x_ref, w_ref, b_ref, out_ref):
    out_ref[...] = (
        jnp.dot(x_ref[...], w_ref[...], preferred_element_type=jnp.float32)
        + b_ref[...]
    )


def _recurrence_body(ih_ref, w_ref, g_ref, bt_ref, out_ref, h_scr, acc):
    t = pl.program_id(0)
    acc[...] = ih_ref[0]

    @pl.when(t > 0)
    def _mm():
        acc[...] += jnp.dot(
            w_ref[...], h_scr[...].astype(jnp.bfloat16),
            preferred_element_type=jnp.float32)

    p = acc[...]
    mu = jnp.mean(p, axis=0, keepdims=True)
    var = jnp.mean((p - mu) * (p - mu), axis=0, keepdims=True)
    hn = jnp.tanh((p - mu) * jax.lax.rsqrt(var + _EPS) * g_ref[...]
                  + bt_ref[...])
    h_scr[...] = hn
    out_ref[0] = hn


def _dense_recurrence(xs_bf, w_ihT, w_hh, bias, ln_gamma, ln_beta):
    w_ihT_bf = w_ihT.astype(jnp.bfloat16)
    w_hh_bf = w_hh.astype(jnp.bfloat16)

    ih_all = pl.pallas_call(
        _ih_matmul_body,
        grid=(_H // 512,),
        in_specs=[
            pl.BlockSpec((_T * _B, _D), lambda j: (0, 0)),
            pl.BlockSpec((_D, 512), lambda j: (0, j)),
            pl.BlockSpec((1, 512), lambda j: (0, j)),
        ],
        out_specs=pl.BlockSpec((_T * _B, 512), lambda j: (0, j)),
        out_shape=jax.ShapeDtypeStruct((_T * _B, _H), jnp.float32),
    )(xs_bf, w_ihT_bf, bias.reshape(1, _H))

    ih3 = ih_all.reshape(_T, _B, _H).transpose(0, 2, 1)

    out = pl.pallas_call(
        _recurrence_body,
        grid=(_T,),
        in_specs=[
            pl.BlockSpec((1, _H, _B), lambda t: (t, 0, 0)),
            pl.BlockSpec((_H, _H), lambda t: (0, 0)),
            pl.BlockSpec((_H, 1), lambda t: (0, 0)),
            pl.BlockSpec((_H, 1), lambda t: (0, 0)),
        ],
        out_specs=pl.BlockSpec((1, _H, _B), lambda t: (t, 0, 0)),
        out_shape=jax.ShapeDtypeStruct((_T, _H, _B), jnp.float32),
        scratch_shapes=[
            pltpu.VMEM((_H, _B), jnp.float32),
            pltpu.VMEM((_H, _B), jnp.float32),
        ],
    )(ih3, w_hh_bf, ln_gamma.reshape(_H, 1), ln_beta.reshape(_H, 1))
    return out.transpose(2, 0, 1)


def kernel(x, ih_indices, ih_values, hh_indices, hh_values,
           bias_ih, bias_hh, ln_gamma, ln_beta):
    w_ihT, w_hh = _densify(ih_indices, ih_values, hh_indices, hh_values)
    xs_bf = x.transpose(1, 0, 2).reshape(_T * _B, _D).astype(jnp.bfloat16)
    bias = bias_ih + bias_hh
    return _dense_recurrence(xs_bf, w_ihT, w_hh, bias, ln_gamma, ln_beta)


# speedup vs baseline: 1.5392x; 1.0674x over previous
import functools

import jax
import jax.numpy as jnp
from jax import lax
from jax.experimental import pallas as pl
from jax.experimental.pallas import tpu as pltpu
from jax.experimental.pallas import tpu_sc as plsc

_B, _T, _D, _H = 64, 8, 4096, 4096
_EPS = 1e-5
_HH = _H // 2

_NNZ = 167772
_NT = 16
_CHUNK_ROWS = 82
_CHUNK = _CHUNK_ROWS * 128
_NNZ_PAD = _NT * _CHUNK
_SLAB_ROWS = 256
_SLAB_ELEMS = _SLAB_ROWS * _H
_SLABS_PER_SC = _H // (2 * _SLAB_ROWS)
_STRIPE = _SLAB_ELEMS // _NT


def _pad_coo(idx, val):
    pad = _NNZ_PAD - _NNZ
    maj = jnp.concatenate([idx[1], jnp.zeros((pad,), jnp.int32)])
    mnr = jnp.concatenate([idx[0], jnp.zeros((pad,), jnp.int32)])
    v = jnp.concatenate([val, jnp.zeros((pad,), jnp.float32)])
    return maj, mnr, v


def _densify_body(maj_ih, mnr_ih, val_ih, maj_hh, mnr_hh, val_hh, zeros_hbm,
                  ih_top, ih_bot, hh_top, hh_bot,
                  pre_v, mnr_v, val_v, off_b, v_b, slab):
    core = lax.axis_index("c")
    sid = lax.axis_index("s")
    core_base = core * (_SLABS_PER_SC * _SLAB_ELEMS)

    for (maj_h, mnr_h, val_h, w_top, w_bot) in (
        (maj_ih, mnr_ih, val_ih, ih_top, ih_bot),
        (maj_hh, mnr_hh, val_hh, hh_top, hh_bot),
    ):
        pltpu.sync_copy(maj_h.at[pl.ds(sid * _CHUNK, _CHUNK)], pre_v)
        pltpu.sync_copy(mnr_h.at[pl.ds(sid * _CHUNK, _CHUNK)], mnr_v)
        pltpu.sync_copy(val_h.at[pl.ds(sid * _CHUNK, _CHUNK)], val_v)

        def _pre(i, _):
            maj16 = pre_v[pl.ds(i * 16, 16)]
            mnr16 = mnr_v[pl.ds(i * 16, 16)]
            pre_v[pl.ds(i * 16, 16)] = maj16 * _H + mnr16 - core_base
            return 0

        lax.fori_loop(0, _CHUNK // 16, _pre, 0)

        for s in range(_SLABS_PER_SC):
            pltpu.sync_copy(zeros_hbm.at[pl.ds(sid * _STRIPE, _STRIPE)],
                            slab.at[pl.ds(sid * _STRIPE, _STRIPE)])
            plsc.subcore_barrier()

            def _mk(j, _):
                for k in range(8):
                    o = j * 128 + k * 16
                    rel = pre_v[pl.ds(o, 16)] - s * _SLAB_ELEMS
                    inb = plsc.bitcast(rel, jnp.uint32) < _SLAB_ELEMS
                    off_b[pl.ds(o, 16)] = rel & (_SLAB_ELEMS - 1)
                    v_b[pl.ds(o, 16)] = jnp.where(
                        inb, val_v[pl.ds(o, 16)], 0.0)
                return 0

            lax.fori_loop(0, _CHUNK_ROWS, _mk, 0)
            pltpu.sync_copy(v_b, slab.at[off_b], add=True)
            plsc.subcore_barrier()
            base = s * _SLAB_ELEMS + sid * _STRIPE

            @pl.when(core == 0)
            def _out_top():
                pltpu.sync_copy(slab.at[pl.ds(sid * _STRIPE, _STRIPE)],
                                w_top.at[pl.ds(base, _STRIPE)])

            @pl.when(core == 1)
            def _out_bot():
                pltpu.sync_copy(slab.at[pl.ds(sid * _STRIPE, _STRIPE)],
                                w_bot.at[pl.ds(base, _STRIPE)])


@jax.jit
def _densify(ih_indices, ih_values, hh_indices, hh_values):
    maj_i, mnr_i, v_i = _pad_coo(ih_indices, ih_values)
    maj_h, mnr_h, v_h = _pad_coo(hh_indices, hh_values)
    zeros = jnp.zeros((_SLAB_ELEMS,), jnp.float32)

    half = jax.ShapeDtypeStruct((_HH * _H,), jnp.float32)
    fn = pl.kernel(
        _densify_body,
        out_type=[half, half, half, half],
        mesh=plsc.VectorSubcoreMesh(
            core_axis_name="c", subcore_axis_name="s"),
        scratch_types=[
            pltpu.VMEM((_CHUNK,), jnp.int32),
            pltpu.VMEM((_CHUNK,), jnp.int32),
            pltpu.VMEM((_CHUNK,), jnp.float32),
            pltpu.VMEM((_CHUNK,), jnp.int32),
            pltpu.VMEM((_CHUNK,), jnp.float32),
            pltpu.VMEM_SHARED((_SLAB_ELEMS,), jnp.float32),
        ],
    )
    ih_top, ih_bot, hh_top, hh_bot = fn(
        maj_i, mnr_i, v_i, maj_h, mnr_h, v_h, zeros)
    to_bf = lambda w: w.astype(jnp.bfloat16).reshape(_HH, _H)
    return (to_bf(ih_top), to_bf(ih_bot)), (to_bf(hh_top), to_bf(hh_bot))


def _ih_matmul_body(x_ref, wt_ref, wb_ref, b_ref, out_ref):
    acc = jnp.dot(x_ref[:, :_HH], wt_ref[...],
                  preferred_element_type=jnp.float32)
    acc += jnp.dot(x_ref[:, _HH:], wb_ref[...],
                   preferred_element_type=jnp.float32)
    out_ref[...] = acc + b_ref[...]


def _recurrence_body(ih_ref, wt_ref, wb_ref, g_ref, bt_ref, out_ref, h_scr):
    t = pl.program_id(0)
    p = ih_ref[0]

    @pl.when(t == 0)
    def _first():
        mu = jnp.mean(p, axis=1, keepdims=True)
        var = jnp.mean((p - mu) * (p - mu), axis=1, keepdims=True)
        hn = jnp.tanh((p - mu) * jax.lax.rsqrt(var + _EPS) * g_ref[...]
                      + bt_ref[...])
        h_scr[...] = hn
        out_ref[0] = hn

    @pl.when(t > 0)
    def _rest():
        h = h_scr[...].astype(jnp.bfloat16)
        q = p + jnp.dot(h[:, :_HH], wt_ref[...],
                        preferred_element_type=jnp.float32)
        q += jnp.dot(h[:, _HH:], wb_ref[...],
                     preferred_element_type=jnp.float32)
        mu = jnp.mean(q, axis=1, keepdims=True)
        var = jnp.mean((q - mu) * (q - mu), axis=1, keepdims=True)
        hn = jnp.tanh((q - mu) * jax.lax.rsqrt(var + _EPS) * g_ref[...]
                      + bt_ref[...])
        h_scr[...] = hn
        out_ref[0] = hn


def _dense_recurrence(xs_bf, w_ih_halves, w_hh_halves, bias,
                      ln_gamma, ln_beta):
    ih_all = pl.pallas_call(
        _ih_matmul_body,
        grid=(_H // 512,),
        in_specs=[
            pl.BlockSpec((_T * _B, _D), lambda j: (0, 0)),
            pl.BlockSpec((_HH, 512), lambda j: (0, j)),
            pl.BlockSpec((_HH, 512), lambda j: (0, j)),
            pl.BlockSpec((1, 512), lambda j: (0, j)),
        ],
        out_specs=pl.BlockSpec((_T * _B, 512), lambda j: (0, j)),
        out_shape=jax.ShapeDtypeStruct((_T * _B, _H), jnp.float32),
    )(xs_bf, w_ih_halves[0], w_ih_halves[1], bias.reshape(1, _H))

    out = pl.pallas_call(
        _recurrence_body,
        grid=(_T,),
        in_specs=[
            pl.BlockSpec((1, _B, _H), lambda t: (t, 0, 0)),
            pl.BlockSpec((_HH, _H), lambda t: (0, 0)),
            pl.BlockSpec((_HH, _H), lambda t: (0, 0)),
            pl.BlockSpec((1, _H), lambda t: (0, 0)),
            pl.BlockSpec((1, _H), lambda t: (0, 0)),
        ],
        out_specs=pl.BlockSpec((1, _B, _H), lambda t: (t, 0, 0)),
        out_shape=jax.ShapeDtypeStruct((_T, _B, _H), jnp.float32),
        scratch_shapes=[
            pltpu.VMEM((_B, _H), jnp.float32),
        ],
    )(ih_all.reshape(_T, _B, _H), w_hh_halves[0], w_hh_halves[1],
      ln_gamma.reshape(1, _H), ln_beta.reshape(1, _H))
    return out.transpose(1, 0, 2)


def kernel(x, ih_indices, ih_values, hh_indices, hh_values,
           bias_ih, bias_hh, ln_gamma, ln_beta):
    w_ih_halves, w_hh_halves = _densify(
        ih_indices, ih_values, hh_indices, hh_values)
    xs_bf = x.transpose(1, 0, 2).reshape(_T * _B, _D).astype(jnp.bfloat16)
    bias = bias_ih + bias_hh
    return _dense_recurrence(xs_bf, w_ih_halves, w_hh_halves, bias,
                             ln_gamma, ln_beta)


# speedup vs baseline: 1.9569x; 1.2714x over previous
import functools

import jax
import jax.numpy as jnp
from jax import lax
from jax.experimental import pallas as pl
from jax.experimental.pallas import tpu as pltpu
from jax.experimental.pallas import tpu_sc as plsc

_B, _T, _D, _H = 64, 8, 4096, 4096
_EPS = 1e-5
_HH = _H // 2

_NNZ = 167772
_NT = 16
_CHUNK_ROWS = 82
_CHUNK = _CHUNK_ROWS * 128
_NNZ_PAD = _NT * _CHUNK
_SLAB_ROWS = 256
_SLAB_ELEMS = _SLAB_ROWS * _H
_SLABS_PER_SC = _H // (2 * _SLAB_ROWS)
_STRIPE = _SLAB_ELEMS // _NT


def _pad_coo(idx, val):
    pad = _NNZ_PAD - _NNZ
    maj = jnp.concatenate([idx[1], jnp.zeros((pad,), jnp.int32)])
    mnr = jnp.concatenate([idx[0], jnp.zeros((pad,), jnp.int32)])
    v = jnp.concatenate([val, jnp.zeros((pad,), jnp.float32)])
    return maj, mnr, v


def _densify_body(maj_h, mnr_h, val_h, zeros_hbm, w_top, w_bot,
                  pre_v, mnr_v, val_v, off_b, v_b, slab):
    core = lax.axis_index("c")
    sid = lax.axis_index("s")
    core_base = core * (_SLABS_PER_SC * _SLAB_ELEMS)

    pltpu.sync_copy(maj_h.at[pl.ds(sid * _CHUNK, _CHUNK)], pre_v)
    pltpu.sync_copy(mnr_h.at[pl.ds(sid * _CHUNK, _CHUNK)], mnr_v)
    pltpu.sync_copy(val_h.at[pl.ds(sid * _CHUNK, _CHUNK)], val_v)

    def _pre(i, _):
        maj16 = pre_v[pl.ds(i * 16, 16)]
        mnr16 = mnr_v[pl.ds(i * 16, 16)]
        pre_v[pl.ds(i * 16, 16)] = maj16 * _H + mnr16 - core_base
        return 0

    lax.fori_loop(0, _CHUNK // 16, _pre, 0)

    for s in range(_SLABS_PER_SC):
        pltpu.sync_copy(zeros_hbm.at[pl.ds(sid * _STRIPE, _STRIPE)],
                        slab.at[pl.ds(sid * _STRIPE, _STRIPE)])
        plsc.subcore_barrier()

        def _mk(j, _):
            for k in range(8):
                o = j * 128 + k * 16
                rel = pre_v[pl.ds(o, 16)] - s * _SLAB_ELEMS
                inb = plsc.bitcast(rel, jnp.uint32) < _SLAB_ELEMS
                off_b[pl.ds(o, 16)] = rel & (_SLAB_ELEMS - 1)
                v_b[pl.ds(o, 16)] = jnp.where(
                    inb, val_v[pl.ds(o, 16)], 0.0)
            return 0

        lax.fori_loop(0, _CHUNK_ROWS, _mk, 0)
        pltpu.sync_copy(v_b, slab.at[off_b], add=True)
        plsc.subcore_barrier()
        base = s * _SLAB_ELEMS + sid * _STRIPE

        @pl.when(core == 0)
        def _out_top():
            pltpu.sync_copy(slab.at[pl.ds(sid * _STRIPE, _STRIPE)],
                            w_top.at[pl.ds(base, _STRIPE)])

        @pl.when(core == 1)
        def _out_bot():
            pltpu.sync_copy(slab.at[pl.ds(sid * _STRIPE, _STRIPE)],
                            w_bot.at[pl.ds(base, _STRIPE)])


def _densify_one(maj, mnr, val, zeros):
    half = jax.ShapeDtypeStruct((_HH * _H,), jnp.float32)
    fn = pl.kernel(
        _densify_body,
        out_type=[half, half],
        mesh=plsc.VectorSubcoreMesh(
            core_axis_name="c", subcore_axis_name="s"),
        scratch_types=[
            pltpu.VMEM((_CHUNK,), jnp.int32),
            pltpu.VMEM((_CHUNK,), jnp.int32),
            pltpu.VMEM((_CHUNK,), jnp.float32),
            pltpu.VMEM((_CHUNK,), jnp.int32),
            pltpu.VMEM((_CHUNK,), jnp.float32),
            pltpu.VMEM_SHARED((_SLAB_ELEMS,), jnp.float32),
        ],
    )
    return fn(maj, mnr, val, zeros)


_RR = 128


def _reformat_body(a_ref, b_ref, c_ref, d_ref, ao, bo, co, do_):
    for i_ref, o_ref in ((a_ref, ao), (b_ref, bo), (c_ref, co), (d_ref, do_)):
        o_ref[...] = i_ref[...].reshape(_RR, _H).astype(jnp.bfloat16)


def _reformat4(a, b, c, d):
    obf = jax.ShapeDtypeStruct((_HH, _H), jnp.bfloat16)
    ispec = pl.BlockSpec((_RR * _H,), lambda j: (j,))
    ospec = pl.BlockSpec((_RR, _H), lambda j: (j, 0))
    return pl.pallas_call(
        _reformat_body,
        grid=(_HH // _RR,),
        in_specs=[ispec] * 4,
        out_specs=[ospec] * 4,
        out_shape=[obf] * 4,
    )(a, b, c, d)


@jax.jit
def _densify(ih_indices, ih_values, hh_indices, hh_values):
    maj_i, mnr_i, v_i = _pad_coo(ih_indices, ih_values)
    maj_h, mnr_h, v_h = _pad_coo(hh_indices, hh_values)
    zeros = jnp.zeros((_SLAB_ELEMS,), jnp.float32)

    ih_top, ih_bot = _densify_one(maj_i, mnr_i, v_i, zeros)
    hh_top, hh_bot = _densify_one(maj_h, mnr_h, v_h, zeros)
    ih_t, ih_b, hh_t, hh_b = _reformat4(ih_top, ih_bot, hh_top, hh_bot)
    return (ih_t, ih_b), (hh_t, hh_b)


def _ih_matmul_body(x_ref, wt_ref, wb_ref, b_ref, out_ref):
    acc = jnp.dot(x_ref[:, :_HH], wt_ref[...],
                  preferred_element_type=jnp.float32)
    acc += jnp.dot(x_ref[:, _HH:], wb_ref[...],
                   preferred_element_type=jnp.float32)
    out_ref[...] = acc + b_ref[...]


def _recurrence_body(ih_ref, wt_ref, wb_ref, g_ref, bt_ref, out_ref, h_scr):
    t = pl.program_id(0)
    p = ih_ref[0]

    @pl.when(t == 0)
    def _first():
        mu = jnp.mean(p, axis=1, keepdims=True)
        var = jnp.mean((p - mu) * (p - mu), axis=1, keepdims=True)
        hn = jnp.tanh((p - mu) * jax.lax.rsqrt(var + _EPS) * g_ref[...]
                      + bt_ref[...])
        h_scr[...] = hn
        out_ref[0] = hn

    @pl.when(t > 0)
    def _rest():
        h = h_scr[...].astype(jnp.bfloat16)
        q = p + jnp.dot(h[:, :_HH], wt_ref[...],
                        preferred_element_type=jnp.float32)
        q += jnp.dot(h[:, _HH:], wb_ref[...],
                     preferred_element_type=jnp.float32)
        mu = jnp.mean(q, axis=1, keepdims=True)
        var = jnp.mean((q - mu) * (q - mu), axis=1, keepdims=True)
        hn = jnp.tanh((q - mu) * jax.lax.rsqrt(var + _EPS) * g_ref[...]
                      + bt_ref[...])
        h_scr[...] = hn
        out_ref[0] = hn


def _dense_recurrence(xs_bf, w_ih_halves, w_hh_halves, bias,
                      ln_gamma, ln_beta):
    ih_all = pl.pallas_call(
        _ih_matmul_body,
        grid=(_H // 512,),
        in_specs=[
            pl.BlockSpec((_T * _B, _D), lambda j: (0, 0)),
            pl.BlockSpec((_HH, 512), lambda j: (0, j)),
            pl.BlockSpec((_HH, 512), lambda j: (0, j)),
            pl.BlockSpec((1, 512), lambda j: (0, j)),
        ],
        out_specs=pl.BlockSpec((_T * _B, 512), lambda j: (0, j)),
        out_shape=jax.ShapeDtypeStruct((_T * _B, _H), jnp.float32),
    )(xs_bf, w_ih_halves[0], w_ih_halves[1], bias.reshape(1, _H))

    out = pl.pallas_call(
        _recurrence_body,
        grid=(_T,),
        in_specs=[
            pl.BlockSpec((1, _B, _H), lambda t: (t, 0, 0)),
            pl.BlockSpec((_HH, _H), lambda t: (0, 0)),
            pl.BlockSpec((_HH, _H), lambda t: (0, 0)),
            pl.BlockSpec((1, _H), lambda t: (0, 0)),
            pl.BlockSpec((1, _H), lambda t: (0, 0)),
        ],
        out_specs=pl.BlockSpec((1, _B, _H), lambda t: (t, 0, 0)),
        out_shape=jax.ShapeDtypeStruct((_T, _B, _H), jnp.float32),
        scratch_shapes=[
            pltpu.VMEM((_B, _H), jnp.float32),
        ],
    )(ih_all.reshape(_T, _B, _H), w_hh_halves[0], w_hh_halves[1],
      ln_gamma.reshape(1, _H), ln_beta.reshape(1, _H))
    return out.transpose(1, 0, 2)


def kernel(x, ih_indices, ih_values, hh_indices, hh_values,
           bias_ih, bias_hh, ln_gamma, ln_beta):
    w_ih_halves, w_hh_halves = _densify(
        ih_indices, ih_values, hh_indices, hh_values)
    xs_bf = x.transpose(1, 0, 2).reshape(_T * _B, _D).astype(jnp.bfloat16)
    bias = bias_ih + bias_hh
    return _dense_recurrence(xs_bf, w_ih_halves, w_hh_halves, bias,
                             ln_gamma, ln_beta)


# speedup vs baseline: 2.0623x; 1.0538x over previous
import functools

import jax
import jax.numpy as jnp
from jax import lax
from jax.experimental import pallas as pl
from jax.experimental.pallas import tpu as pltpu
from jax.experimental.pallas import tpu_sc as plsc

_B, _T, _D, _H = 64, 8, 4096, 4096
_EPS = 1e-5
_HH = _H // 2

_NNZ = 167772
_NT = 16
_CHUNK_ROWS = 82
_CHUNK = _CHUNK_ROWS * 128
_NNZ_PAD = _NT * _CHUNK
_SLAB_ROWS = 256
_SLAB_ELEMS = _SLAB_ROWS * _H
_SLABS_PER_SC = _H // (2 * _SLAB_ROWS)
_STRIPE = _SLAB_ELEMS // _NT


def _pad_coo(idx, val):
    pad = _NNZ_PAD - _NNZ
    maj = jnp.concatenate([idx[1], jnp.zeros((pad,), jnp.int32)])
    mnr = jnp.concatenate([idx[0], jnp.zeros((pad,), jnp.int32)])
    v = jnp.concatenate([val, jnp.zeros((pad,), jnp.float32)])
    return maj, mnr, v


def _densify_body(maj_h, mnr_h, val_h, zeros_hbm, w_top, w_bot,
                  pre_v, mnr_v, val_v, off_b, v_b, slab):
    core = lax.axis_index("c")
    sid = lax.axis_index("s")
    core_base = core * (_SLABS_PER_SC * _SLAB_ELEMS)

    pltpu.sync_copy(maj_h.at[pl.ds(sid * _CHUNK, _CHUNK)], pre_v)
    pltpu.sync_copy(mnr_h.at[pl.ds(sid * _CHUNK, _CHUNK)], mnr_v)
    pltpu.sync_copy(val_h.at[pl.ds(sid * _CHUNK, _CHUNK)], val_v)

    def _pre(i, _):
        maj16 = pre_v[pl.ds(i * 16, 16)]
        mnr16 = mnr_v[pl.ds(i * 16, 16)]
        pre_v[pl.ds(i * 16, 16)] = maj16 * _H + mnr16 - core_base
        return 0

    lax.fori_loop(0, _CHUNK // 16, _pre, 0)

    for s in range(_SLABS_PER_SC):
        pltpu.sync_copy(zeros_hbm.at[pl.ds(sid * _STRIPE, _STRIPE)],
                        slab.at[pl.ds(sid * _STRIPE, _STRIPE)])
        plsc.subcore_barrier()

        def _mk(j, _):
            for k in range(8):
                o = j * 128 + k * 16
                rel = pre_v[pl.ds(o, 16)] - s * _SLAB_ELEMS
                inb = plsc.bitcast(rel, jnp.uint32) < _SLAB_ELEMS
                off_b[pl.ds(o, 16)] = rel & (_SLAB_ELEMS - 1)
                v_b[pl.ds(o, 16)] = jnp.where(
                    inb, val_v[pl.ds(o, 16)], 0.0)
            return 0

        lax.fori_loop(0, _CHUNK_ROWS, _mk, 0)
        pltpu.sync_copy(v_b, slab.at[off_b], add=True)
        plsc.subcore_barrier()
        base = s * _SLAB_ELEMS + sid * _STRIPE

        @pl.when(core == 0)
        def _out_top():
            pltpu.sync_copy(slab.at[pl.ds(sid * _STRIPE, _STRIPE)],
                            w_top.at[pl.ds(base, _STRIPE)])

        @pl.when(core == 1)
        def _out_bot():
            pltpu.sync_copy(slab.at[pl.ds(sid * _STRIPE, _STRIPE)],
                            w_bot.at[pl.ds(base, _STRIPE)])


def _densify_one(maj, mnr, val, zeros):
    half = jax.ShapeDtypeStruct((_HH * _H,), jnp.float32)
    fn = pl.kernel(
        _densify_body,
        out_type=[half, half],
        mesh=plsc.VectorSubcoreMesh(
            core_axis_name="c", subcore_axis_name="s"),
        scratch_types=[
            pltpu.VMEM((_CHUNK,), jnp.int32),
            pltpu.VMEM((_CHUNK,), jnp.int32),
            pltpu.VMEM((_CHUNK,), jnp.float32),
            pltpu.VMEM((_CHUNK,), jnp.int32),
            pltpu.VMEM((_CHUNK,), jnp.float32),
            pltpu.VMEM_SHARED((_SLAB_ELEMS,), jnp.float32),
        ],
    )
    return fn(maj, mnr, val, zeros)


_RR = 128


def _reformat_body(a_ref, b_ref, ao, bo):
    for i_ref, o_ref in ((a_ref, ao), (b_ref, bo)):
        o_ref[...] = i_ref[...].reshape(_RR, _H).astype(jnp.bfloat16)


def _reformat2(a, b):
    obf = jax.ShapeDtypeStruct((_HH, _H), jnp.bfloat16)
    ispec = pl.BlockSpec((_RR * _H,), lambda j: (j,))
    ospec = pl.BlockSpec((_RR, _H), lambda j: (j, 0))
    return pl.pallas_call(
        _reformat_body,
        grid=(_HH // _RR,),
        in_specs=[ispec] * 2,
        out_specs=[ospec] * 2,
        out_shape=[obf] * 2,
    )(a, b)


@jax.jit
def _densify(ih_indices, ih_values, hh_indices, hh_values):
    maj_i, mnr_i, v_i = _pad_coo(ih_indices, ih_values)
    maj_h, mnr_h, v_h = _pad_coo(hh_indices, hh_values)
    zeros = jnp.zeros((_SLAB_ELEMS,), jnp.float32)

    ih_top, ih_bot = _densify_one(maj_i, mnr_i, v_i, zeros)
    hh_top, hh_bot = _densify_one(maj_h, mnr_h, v_h, zeros)
    return _reformat2(ih_top, ih_bot), _reformat2(hh_top, hh_bot)


def _ih_matmul_body(x_ref, wt_ref, wb_ref, b_ref, out_ref):
    acc = jnp.dot(x_ref[:, :_HH], wt_ref[...],
                  preferred_element_type=jnp.float32)
    acc += jnp.dot(x_ref[:, _HH:], wb_ref[...],
                   preferred_element_type=jnp.float32)
    out_ref[...] = acc + b_ref[...]


def _recurrence_body(ih_ref, wt_ref, wb_ref, g_ref, bt_ref, out_ref, h_scr):
    t = pl.program_id(0)
    p = ih_ref[0]

    @pl.when(t == 0)
    def _first():
        mu = jnp.mean(p, axis=1, keepdims=True)
        var = jnp.mean((p - mu) * (p - mu), axis=1, keepdims=True)
        hn = jnp.tanh((p - mu) * jax.lax.rsqrt(var + _EPS) * g_ref[...]
                      + bt_ref[...])
        h_scr[...] = hn
        out_ref[0] = hn

    @pl.when(t > 0)
    def _rest():
        h = h_scr[...].astype(jnp.bfloat16)
        q = p + jnp.dot(h[:, :_HH], wt_ref[...],
                        preferred_element_type=jnp.float32)
        q += jnp.dot(h[:, _HH:], wb_ref[...],
                     preferred_element_type=jnp.float32)
        mu = jnp.mean(q, axis=1, keepdims=True)
        var = jnp.mean((q - mu) * (q - mu), axis=1, keepdims=True)
        hn = jnp.tanh((q - mu) * jax.lax.rsqrt(var + _EPS) * g_ref[...]
                      + bt_ref[...])
        h_scr[...] = hn
        out_ref[0] = hn


def _dense_recurrence(xs_bf, w_ih_halves, w_hh_halves, bias,
                      ln_gamma, ln_beta):
    ih_all = pl.pallas_call(
        _ih_matmul_body,
        grid=(_H // 512,),
        in_specs=[
            pl.BlockSpec((_T * _B, _D), lambda j: (0, 0)),
            pl.BlockSpec((_HH, 512), lambda j: (0, j)),
            pl.BlockSpec((_HH, 512), lambda j: (0, j)),
            pl.BlockSpec((1, 512), lambda j: (0, j)),
        ],
        out_specs=pl.BlockSpec((_T * _B, 512), lambda j: (0, j)),
        out_shape=jax.ShapeDtypeStruct((_T * _B, _H), jnp.float32),
    )(xs_bf, w_ih_halves[0], w_ih_halves[1], bias.reshape(1, _H))

    out = pl.pallas_call(
        _recurrence_body,
        grid=(_T,),
        in_specs=[
            pl.BlockSpec((1, _B, _H), lambda t: (t, 0, 0)),
            pl.BlockSpec((_HH, _H), lambda t: (0, 0)),
            pl.BlockSpec((_HH, _H), lambda t: (0, 0)),
            pl.BlockSpec((1, _H), lambda t: (0, 0)),
            pl.BlockSpec((1, _H), lambda t: (0, 0)),
        ],
        out_specs=pl.BlockSpec((1, _B, _H), lambda t: (t, 0, 0)),
        out_shape=jax.ShapeDtypeStruct((_T, _B, _H), jnp.float32),
        scratch_shapes=[
            pltpu.VMEM((_B, _H), jnp.float32),
        ],
    )(ih_all.reshape(_T, _B, _H), w_hh_halves[0], w_hh_halves[1],
      ln_gamma.reshape(1, _H), ln_beta.reshape(1, _H))
    return out.transpose(1, 0, 2)


def kernel(x, ih_indices, ih_values, hh_indices, hh_values,
           bias_ih, bias_hh, ln_gamma, ln_beta):
    w_ih_halves, w_hh_halves = _densify(
        ih_indices, ih_values, hh_indices, hh_values)
    xs_bf = x.transpose(1, 0, 2).reshape(_T * _B, _D).astype(jnp.bfloat16)
    bias = bias_ih + bias_hh
    return _dense_recurrence(xs_bf, w_ih_halves, w_hh_halves, bias,
                             ln_gamma, ln_beta)


# speedup vs baseline: 2.1744x; 1.0543x over previous
import functools

import jax
import jax.numpy as jnp
from jax import lax
from jax.experimental import pallas as pl
from jax.experimental.pallas import tpu as pltpu
from jax.experimental.pallas import tpu_sc as plsc

_B, _T, _D, _H = 64, 8, 4096, 4096
_EPS = 1e-5
_HH = _H // 2

_NNZ = 167772
_NT = 16
_CHUNK_ROWS = 82
_CHUNK = _CHUNK_ROWS * 128
_NNZ_PAD = _NT * _CHUNK
_SLAB_ROWS = 256
_SLAB_ELEMS = _SLAB_ROWS * _H
_SLABS_PER_SC = _H // (2 * _SLAB_ROWS)
_STRIPE = _SLAB_ELEMS // _NT


def _pad_coo(idx, val):
    pad = _NNZ_PAD - _NNZ
    maj = jnp.concatenate([idx[1], jnp.zeros((pad,), jnp.int32)])
    mnr = jnp.concatenate([idx[0], jnp.zeros((pad,), jnp.int32)])
    v = jnp.concatenate([val, jnp.zeros((pad,), jnp.float32)])
    return maj, mnr, v


def _densify_body(maj_h, mnr_h, val_h, zeros_hbm, w_top, w_bot,
                  pre_v, mnr_v, val_v, off_b, v_b, slab):
    core = lax.axis_index("c")
    sid = lax.axis_index("s")
    core_base = core * (_SLABS_PER_SC * _SLAB_ELEMS)

    pltpu.sync_copy(maj_h.at[pl.ds(sid * _CHUNK, _CHUNK)], pre_v)
    pltpu.sync_copy(mnr_h.at[pl.ds(sid * _CHUNK, _CHUNK)], mnr_v)
    pltpu.sync_copy(val_h.at[pl.ds(sid * _CHUNK, _CHUNK)], val_v)

    def _pre(i, _):
        maj16 = pre_v[pl.ds(i * 16, 16)]
        mnr16 = mnr_v[pl.ds(i * 16, 16)]
        pre_v[pl.ds(i * 16, 16)] = maj16 * _H + mnr16 - core_base
        return 0

    lax.fori_loop(0, _CHUNK // 16, _pre, 0)

    for s in range(_SLABS_PER_SC):
        pltpu.sync_copy(zeros_hbm.at[pl.ds(sid * _STRIPE, _STRIPE)],
                        slab.at[pl.ds(sid * _STRIPE, _STRIPE)])
        plsc.subcore_barrier()

        def _mk(j, _):
            for k in range(8):
                o = j * 128 + k * 16
                rel = pre_v[pl.ds(o, 16)] - s * _SLAB_ELEMS
                inb = plsc.bitcast(rel, jnp.uint32) < _SLAB_ELEMS
                off_b[pl.ds(o, 16)] = rel & (_SLAB_ELEMS - 1)
                v_b[pl.ds(o, 16)] = jnp.where(
                    inb, val_v[pl.ds(o, 16)], 0.0)
            return 0

        lax.fori_loop(0, _CHUNK_ROWS, _mk, 0)
        pltpu.sync_copy(v_b, slab.at[off_b], add=True)
        plsc.subcore_barrier()
        base = s * _SLAB_ELEMS + sid * _STRIPE

        @pl.when(core == 0)
        def _out_top():
            pltpu.sync_copy(slab.at[pl.ds(sid * _STRIPE, _STRIPE)],
                            w_top.at[pl.ds(base, _STRIPE)])

        @pl.when(core == 1)
        def _out_bot():
            pltpu.sync_copy(slab.at[pl.ds(sid * _STRIPE, _STRIPE)],
                            w_bot.at[pl.ds(base, _STRIPE)])


def _densify_one(maj, mnr, val, zeros):
    half = jax.ShapeDtypeStruct((_HH * _H,), jnp.float32)
    fn = pl.kernel(
        _densify_body,
        out_type=[half, half],
        mesh=plsc.VectorSubcoreMesh(
            core_axis_name="c", subcore_axis_name="s"),
        scratch_types=[
            pltpu.VMEM((_CHUNK,), jnp.int32),
            pltpu.VMEM((_CHUNK,), jnp.int32),
            pltpu.VMEM((_CHUNK,), jnp.float32),
            pltpu.VMEM((_CHUNK,), jnp.int32),
            pltpu.VMEM((_CHUNK,), jnp.float32),
            pltpu.VMEM_SHARED((_SLAB_ELEMS,), jnp.float32),
        ],
    )
    return fn(maj, mnr, val, zeros)


_RR = 128


def _reformat_body(a_ref, b_ref, ao, bo):
    for i_ref, o_ref in ((a_ref, ao), (b_ref, bo)):
        o_ref[...] = i_ref[...].reshape(_RR, _H).astype(jnp.bfloat16)


def _reformat2(a, b):
    obf = jax.ShapeDtypeStruct((_HH, _H), jnp.bfloat16)
    ispec = pl.BlockSpec((_RR * _H,), lambda j: (j,))
    ospec = pl.BlockSpec((_RR, _H), lambda j: (j, 0))
    return pl.pallas_call(
        _reformat_body,
        grid=(_HH // _RR,),
        in_specs=[ispec] * 2,
        out_specs=[ospec] * 2,
        out_shape=[obf] * 2,
    )(a, b)


@jax.jit
def _densify(ih_indices, ih_values, hh_indices, hh_values):
    maj_i, mnr_i, v_i = _pad_coo(ih_indices, ih_values)
    maj_h, mnr_h, v_h = _pad_coo(hh_indices, hh_values)
    zeros = jnp.zeros((_SLAB_ELEMS,), jnp.float32)

    ih_top, ih_bot = _densify_one(maj_i, mnr_i, v_i, zeros)
    hh_top, hh_bot = _densify_one(maj_h, mnr_h, v_h, zeros)
    return (ih_top, ih_bot), _reformat2(hh_top, hh_bot)


_KI = 256


def _ih_matmul_body(x_ref, wt_ref, wb_ref, b_ref, out_ref, acc):
    c = pl.program_id(0)

    @pl.when(c == 0)
    def _init():
        acc[...] = jnp.zeros_like(acc)

    x_sl = x_ref[:, pl.ds(c * _KI, _KI)]

    @pl.when(c < _HH // _KI)
    def _top():
        w = wt_ref[...].reshape(_KI, _H).astype(jnp.bfloat16)
        acc[...] += jnp.dot(x_sl, w, preferred_element_type=jnp.float32)

    @pl.when(c >= _HH // _KI)
    def _bot():
        w = wb_ref[...].reshape(_KI, _H).astype(jnp.bfloat16)
        acc[...] += jnp.dot(x_sl, w, preferred_element_type=jnp.float32)

    @pl.when(c == _H // _KI - 1)
    def _fin():
        out_ref[...] = acc[...] + b_ref[...]


def _recurrence_body(ih_ref, wt_ref, wb_ref, g_ref, bt_ref, out_ref, h_scr):
    t = pl.program_id(0)
    p = ih_ref[0]

    @pl.when(t == 0)
    def _first():
        mu = jnp.mean(p, axis=1, keepdims=True)
        var = jnp.mean((p - mu) * (p - mu), axis=1, keepdims=True)
        hn = jnp.tanh((p - mu) * jax.lax.rsqrt(var + _EPS) * g_ref[...]
                      + bt_ref[...])
        h_scr[...] = hn
        out_ref[0] = hn

    @pl.when(t > 0)
    def _rest():
        h = h_scr[...].astype(jnp.bfloat16)
        q = p + jnp.dot(h[:, :_HH], wt_ref[...],
                        preferred_element_type=jnp.float32)
        q += jnp.dot(h[:, _HH:], wb_ref[...],
                     preferred_element_type=jnp.float32)
        mu = jnp.mean(q, axis=1, keepdims=True)
        var = jnp.mean((q - mu) * (q - mu), axis=1, keepdims=True)
        hn = jnp.tanh((q - mu) * jax.lax.rsqrt(var + _EPS) * g_ref[...]
                      + bt_ref[...])
        h_scr[...] = hn
        out_ref[0] = hn


def _dense_recurrence(xs_bf, w_ih_halves, w_hh_halves, bias,
                      ln_gamma, ln_beta):
    nb = _HH // _KI
    ih_all = pl.pallas_call(
        _ih_matmul_body,
        grid=(_H // _KI,),
        in_specs=[
            pl.BlockSpec((_T * _B, _D), lambda c: (0, 0)),
            pl.BlockSpec((_KI * _H,), lambda c: (jnp.minimum(c, nb - 1),)),
            pl.BlockSpec((_KI * _H,),
                         lambda c: (jnp.maximum(c - nb, 0),)),
            pl.BlockSpec((1, _H), lambda c: (0, 0)),
        ],
        out_specs=pl.BlockSpec((_T * _B, _H), lambda c: (0, 0)),
        out_shape=jax.ShapeDtypeStruct((_T * _B, _H), jnp.float32),
        scratch_shapes=[pltpu.VMEM((_T * _B, _H), jnp.float32)],
    )(xs_bf, w_ih_halves[0], w_ih_halves[1], bias.reshape(1, _H))

    out = pl.pallas_call(
        _recurrence_body,
        grid=(_T,),
        in_specs=[
            pl.BlockSpec((1, _B, _H), lambda t: (t, 0, 0)),
            pl.BlockSpec((_HH, _H), lambda t: (0, 0)),
            pl.BlockSpec((_HH, _H), lambda t: (0, 0)),
            pl.BlockSpec((1, _H), lambda t: (0, 0)),
            pl.BlockSpec((1, _H), lambda t: (0, 0)),
        ],
        out_specs=pl.BlockSpec((1, _B, _H), lambda t: (t, 0, 0)),
        out_shape=jax.ShapeDtypeStruct((_T, _B, _H), jnp.float32),
        scratch_shapes=[
            pltpu.VMEM((_B, _H), jnp.float32),
        ],
    )(ih_all.reshape(_T, _B, _H), w_hh_halves[0], w_hh_halves[1],
      ln_gamma.reshape(1, _H), ln_beta.reshape(1, _H))
    return out.transpose(1, 0, 2)


def kernel(x, ih_indices, ih_values, hh_indices, hh_values,
           bias_ih, bias_hh, ln_gamma, ln_beta):
    w_ih_halves, w_hh_halves = _densify(
        ih_indices, ih_values, hh_indices, hh_values)
    xs_bf = x.transpose(1, 0, 2).reshape(_T * _B, _D).astype(jnp.bfloat16)
    bias = bias_ih + bias_hh
    return _dense_recurrence(xs_bf, w_ih_halves, w_hh_halves, bias,
                             ln_gamma, ln_beta)


# speedup vs baseline: 2.1744x; 1.0000x over previous
import functools

import jax
import jax.numpy as jnp
from jax import lax
from jax.experimental import pallas as pl
from jax.experimental.pallas import tpu as pltpu
from jax.experimental.pallas import tpu_sc as plsc

_B, _T, _D, _H = 64, 8, 4096, 4096
_EPS = 1e-5
_HH = _H // 2

_NNZ = 167772
_NT = 16
_CHUNK_ROWS = 82
_CHUNK = _CHUNK_ROWS * 128
_NNZ_PAD = _NT * _CHUNK
_SLAB_ROWS = 256
_SLAB_ELEMS = _SLAB_ROWS * _H
_SLABS_PER_SC = _H // (2 * _SLAB_ROWS)
_STRIPE = _SLAB_ELEMS // _NT


def _pad_coo(idx, val):
    pad = _NNZ_PAD - _NNZ
    maj = jnp.concatenate([idx[1], jnp.zeros((pad,), jnp.int32)])
    mnr = jnp.concatenate([idx[0], jnp.zeros((pad,), jnp.int32)])
    v = jnp.concatenate([val, jnp.zeros((pad,), jnp.float32)])
    return maj, mnr, v


def _densify_body(maj_h, mnr_h, val_h, zeros_hbm, w_top, w_bot,
                  pre_v, mnr_v, val_v, off_b, v_b, slab):
    core = lax.axis_index("c")
    sid = lax.axis_index("s")
    core_base = core * (_SLABS_PER_SC * _SLAB_ELEMS)

    pltpu.sync_copy(maj_h.at[pl.ds(sid * _CHUNK, _CHUNK)], pre_v)
    pltpu.sync_copy(mnr_h.at[pl.ds(sid * _CHUNK, _CHUNK)], mnr_v)
    pltpu.sync_copy(val_h.at[pl.ds(sid * _CHUNK, _CHUNK)], val_v)

    def _pre(i, _):
        maj16 = pre_v[pl.ds(i * 16, 16)]
        mnr16 = mnr_v[pl.ds(i * 16, 16)]
        pre_v[pl.ds(i * 16, 16)] = maj16 * _H + mnr16 - core_base
        return 0

    lax.fori_loop(0, _CHUNK // 16, _pre, 0)

    for s in range(_SLABS_PER_SC):
        pltpu.sync_copy(zeros_hbm.at[pl.ds(sid * _STRIPE, _STRIPE)],
                        slab.at[pl.ds(sid * _STRIPE, _STRIPE)])
        plsc.subcore_barrier()

        def _mk(j, _):
            for k in range(8):
                o = j * 128 + k * 16
                rel = pre_v[pl.ds(o, 16)] - s * _SLAB_ELEMS
                inb = plsc.bitcast(rel, jnp.uint32) < _SLAB_ELEMS
                off_b[pl.ds(o, 16)] = rel & (_SLAB_ELEMS - 1)
                v_b[pl.ds(o, 16)] = jnp.where(
                    inb, val_v[pl.ds(o, 16)], 0.0)
            return 0

        lax.fori_loop(0, _CHUNK_ROWS, _mk, 0)
        pltpu.sync_copy(v_b, slab.at[off_b], add=True)
        plsc.subcore_barrier()
        base = s * _SLAB_ELEMS + sid * _STRIPE

        @pl.when(core == 0)
        def _out_top():
            pltpu.sync_copy(slab.at[pl.ds(sid * _STRIPE, _STRIPE)],
                            w_top.at[pl.ds(base, _STRIPE)])

        @pl.when(core == 1)
        def _out_bot():
            pltpu.sync_copy(slab.at[pl.ds(sid * _STRIPE, _STRIPE)],
                            w_bot.at[pl.ds(base, _STRIPE)])


def _densify_one(maj, mnr, val, zeros):
    half = jax.ShapeDtypeStruct((_HH * _H,), jnp.float32)
    fn = pl.kernel(
        _densify_body,
        out_type=[half, half],
        mesh=plsc.VectorSubcoreMesh(
            core_axis_name="c", subcore_axis_name="s"),
        scratch_types=[
            pltpu.VMEM((_CHUNK,), jnp.int32),
            pltpu.VMEM((_CHUNK,), jnp.int32),
            pltpu.VMEM((_CHUNK,), jnp.float32),
            pltpu.VMEM((_CHUNK,), jnp.int32),
            pltpu.VMEM((_CHUNK,), jnp.float32),
            pltpu.VMEM_SHARED((_SLAB_ELEMS,), jnp.float32),
        ],
    )
    return fn(maj, mnr, val, zeros)


_RR = 128


def _reformat_body(a_ref, b_ref, ao, bo):
    for i_ref, o_ref in ((a_ref, ao), (b_ref, bo)):
        o_ref[...] = i_ref[...].reshape(_RR, _H).astype(jnp.bfloat16)


def _reformat2(a, b):
    obf = jax.ShapeDtypeStruct((_HH, _H), jnp.bfloat16)
    ispec = pl.BlockSpec((_RR * _H,), lambda j: (j,))
    ospec = pl.BlockSpec((_RR, _H), lambda j: (j, 0))
    return pl.pallas_call(
        _reformat_body,
        grid=(_HH // _RR,),
        in_specs=[ispec] * 2,
        out_specs=[ospec] * 2,
        out_shape=[obf] * 2,
    )(a, b)


_KI = 256


def _ih_matmul_body(x_ref, wt_ref, wb_ref, b_ref, out_ref, acc):
    c = pl.program_id(0)

    @pl.when(c == 0)
    def _init():
        acc[...] = jnp.zeros_like(acc)

    x_sl = x_ref[:, pl.ds(c * _KI, _KI)]

    @pl.when(c < _HH // _KI)
    def _top():
        w = wt_ref[...].reshape(_KI, _H).astype(jnp.bfloat16)
        acc[...] += jnp.dot(x_sl, w, preferred_element_type=jnp.float32)

    @pl.when(c >= _HH // _KI)
    def _bot():
        w = wb_ref[...].reshape(_KI, _H).astype(jnp.bfloat16)
        acc[...] += jnp.dot(x_sl, w, preferred_element_type=jnp.float32)

    @pl.when(c == _H // _KI - 1)
    def _fin():
        out_ref[...] = acc[...] + b_ref[...]


def _recurrence_body(ih_ref, wt_ref, wb_ref, g_ref, bt_ref, out_ref, h_scr):
    t = pl.program_id(0)
    p = ih_ref[0]

    @pl.when(t == 0)
    def _first():
        mu = jnp.mean(p, axis=1, keepdims=True)
        var = jnp.mean((p - mu) * (p - mu), axis=1, keepdims=True)
        hn = jnp.tanh((p - mu) * jax.lax.rsqrt(var + _EPS) * g_ref[...]
                      + bt_ref[...])
        h_scr[...] = hn
        out_ref[0] = hn

    @pl.when(t > 0)
    def _rest():
        h = h_scr[...].astype(jnp.bfloat16)
        q = p + jnp.dot(h[:, :_HH], wt_ref[...],
                        preferred_element_type=jnp.float32)
        q += jnp.dot(h[:, _HH:], wb_ref[...],
                     preferred_element_type=jnp.float32)
        mu = jnp.mean(q, axis=1, keepdims=True)
        var = jnp.mean((q - mu) * (q - mu), axis=1, keepdims=True)
        hn = jnp.tanh((q - mu) * jax.lax.rsqrt(var + _EPS) * g_ref[...]
                      + bt_ref[...])
        h_scr[...] = hn
        out_ref[0] = hn


def _ih_matmul(xs_bf, w_ih_halves, bias):
    nb = _HH // _KI
    ih_all = pl.pallas_call(
        _ih_matmul_body,
        grid=(_H // _KI,),
        in_specs=[
            pl.BlockSpec((_T * _B, _D), lambda c: (0, 0)),
            pl.BlockSpec((_KI * _H,), lambda c: (jnp.minimum(c, nb - 1),)),
            pl.BlockSpec((_KI * _H,),
                         lambda c: (jnp.maximum(c - nb, 0),)),
            pl.BlockSpec((1, _H), lambda c: (0, 0)),
        ],
        out_specs=pl.BlockSpec((_T * _B, _H), lambda c: (0, 0)),
        out_shape=jax.ShapeDtypeStruct((_T * _B, _H), jnp.float32),
        scratch_shapes=[pltpu.VMEM((_T * _B, _H), jnp.float32)],
    )(xs_bf, w_ih_halves[0], w_ih_halves[1], bias.reshape(1, _H))
    return ih_all


def _recurrence(ih_all, w_hh_halves, ln_gamma, ln_beta):
    out = pl.pallas_call(
        _recurrence_body,
        grid=(_T,),
        in_specs=[
            pl.BlockSpec((1, _B, _H), lambda t: (t, 0, 0)),
            pl.BlockSpec((_HH, _H), lambda t: (0, 0)),
            pl.BlockSpec((_HH, _H), lambda t: (0, 0)),
            pl.BlockSpec((1, _H), lambda t: (0, 0)),
            pl.BlockSpec((1, _H), lambda t: (0, 0)),
        ],
        out_specs=pl.BlockSpec((1, _B, _H), lambda t: (t, 0, 0)),
        out_shape=jax.ShapeDtypeStruct((_T, _B, _H), jnp.float32),
        scratch_shapes=[
            pltpu.VMEM((_B, _H), jnp.float32),
        ],
    )(ih_all.reshape(_T, _B, _H), w_hh_halves[0], w_hh_halves[1],
      ln_gamma.reshape(1, _H), ln_beta.reshape(1, _H))
    return out.transpose(1, 0, 2)


def kernel(x, ih_indices, ih_values, hh_indices, hh_values,
           bias_ih, bias_hh, ln_gamma, ln_beta):
    maj_i, mnr_i, v_i = _pad_coo(ih_indices, ih_values)
    maj_h, mnr_h, v_h = _pad_coo(hh_indices, hh_values)
    zeros = jnp.zeros((_SLAB_ELEMS,), jnp.float32)
    xs_bf = x.transpose(1, 0, 2).reshape(_T * _B, _D).astype(jnp.bfloat16)
    bias = bias_ih + bias_hh

    ih_halves = _densify_one(maj_i, mnr_i, v_i, zeros)
    ih_all = _ih_matmul(xs_bf, ih_halves, bias)
    hh_top, hh_bot = _densify_one(maj_h, mnr_h, v_h, zeros)
    w_hh_halves = _reformat2(hh_top, hh_bot)
    return _recurrence(ih_all, w_hh_halves, ln_gamma, ln_beta)
